# Initial kernel scaffold; baseline (speedup 1.0000x reference)
import jax
import jax.numpy as jnp
from jax.experimental import pallas as pl


def kernel(x, y):
    raise NotImplementedError("write your pallas kernel here")


# speedup vs baseline: 13.1035x; 13.1035x over previous
import math

import jax
import jax.numpy as jnp
from jax.experimental import pallas as pl
from jax.experimental.pallas import tpu as pltpu

MASS_CONST = 0.1
IGNORE_CONST = 0

BLK_C = 512


def _main_body(x_ref, y_ref, g_ref, out_ref, m_ref, s_ref, sx_ref, gv_ref,
               *, B, C, T, eps, pme, kconst, use_g):
    b = pl.program_id(0)
    cb = pl.program_id(1)
    ncb = pl.num_programs(1)

    @pl.when(cb == 0)
    def _init():
        m_ref[...] = jnp.full((1, T), -1e37, dtype=jnp.float32)
        s_ref[...] = jnp.zeros((1, T), dtype=jnp.float32)
        sx_ref[...] = jnp.zeros((1, T), dtype=jnp.float32)
        if not use_g:
            gv_ref[...] = jnp.zeros((1, T), dtype=jnp.float32)

    xb = x_ref[0]
    bm = jnp.max(xb, axis=0, keepdims=True)
    m_old = m_ref[...]
    m_new = jnp.maximum(m_old, bm)
    s_ref[...] = (s_ref[...] * jnp.exp(m_old - m_new)
                  + jnp.sum(jnp.exp(xb - m_new), axis=0, keepdims=True))
    sx_ref[...] = sx_ref[...] + jnp.sum(xb, axis=0, keepdims=True)
    m_ref[...] = m_new

    if not use_g:
        row_ids = cb * BLK_C + jax.lax.broadcasted_iota(jnp.int32, (BLK_C, T), 0)
        hit = row_ids == y_ref[0]
        gv_ref[...] = gv_ref[...] + jnp.sum(
            jnp.where(hit, xb, 0.0), axis=0, keepdims=True)

    @pl.when(cb == ncb - 1)
    def _finalize():
        lse = m_ref[...] + jnp.log(s_ref[...])
        gv = g_ref[0] if use_g else gv_ref[...]
        valid = y_ref[0] != IGNORE_CONST
        contrib = jnp.where(
            valid, kconst - eps * sx_ref[...] + lse - pme * gv, 0.0)
        part = jnp.sum(contrib) * (1.0 / B)

        @pl.when(b == 0)
        def _():
            out_ref[...] = part.reshape(1, 1)

        @pl.when(b != 0)
        def _():
            out_ref[...] = out_ref[...] + part.reshape(1, 1)


def _run_main(x, y3, g3, *, interpret=False):
    B, C, T = x.shape
    eps = MASS_CONST / (C - 1)
    p = 1.0 - MASS_CONST
    kconst = p * math.log(p) + MASS_CONST * math.log(eps)
    pme = p - eps
    use_g = g3 is not None
    ncb = C // BLK_C

    import functools
    body = functools.partial(_main_body, B=B, C=C, T=T, eps=eps, pme=pme,
                             kconst=kconst, use_g=use_g)

    in_specs = [
        pl.BlockSpec((1, BLK_C, T), lambda b, cb: (b, cb, 0)),
        pl.BlockSpec((1, 1, T), lambda b, cb: (b, 0, 0)),
        pl.BlockSpec((1, 1, T), lambda b, cb: (b, 0, 0)),
    ]
    args = [x, y3, g3 if use_g else jnp.zeros((B, 1, T), jnp.float32)]

    out = pl.pallas_call(
        body,
        grid=(B, ncb),
        in_specs=in_specs,
        out_specs=pl.BlockSpec((1, 1), lambda b, cb: (0, 0)),
        out_shape=jax.ShapeDtypeStruct((1, 1), jnp.float32),
        scratch_shapes=[
            pltpu.VMEM((1, T), jnp.float32),
            pltpu.VMEM((1, T), jnp.float32),
            pltpu.VMEM((1, T), jnp.float32),
            pltpu.VMEM((1, T), jnp.float32),
        ],
        interpret=interpret,
    )(*args)
    return out[0, 0]


def kernel(x, y):
    B, C, T = x.shape
    y32 = y.astype(jnp.int32).reshape(B, 1, T)
    return _run_main(x, y32, None)
